# baseline (speedup 1.0000x reference)
import functools

import jax
import jax.numpy as jnp
from jax import lax
from jax.experimental import pallas as pl
from jax.experimental.pallas import tpu as pltpu
from jax.experimental.pallas import tpu_sc as plsc

N_NODES = 10000
N_EDGES = 320000
F = 128

NC = 2
NS = 16
NW = NC * NS
K = 128
TOTAL_CHUNKS = N_EDGES // K
BASE_CHUNKS = TOTAL_CHUNKS // NW
EXTRA = TOTAL_CHUNKS - BASE_CHUNKS * NW
N_PAD = 10240
ROWS_PER_TILE = N_PAD // NS
ZCOPY = ROWS_PER_TILE // K


def _linear_body(h_ref, norm_ref, w_ref, o_ref):
    o_ref[...] = norm_ref[...] * jnp.dot(
        h_ref[...], w_ref[...], preferred_element_type=jnp.float32)


def _tc_linear(h, norm, weight):
    blk = 2000
    grid = (N_NODES // blk,)
    return pl.pallas_call(
        _linear_body,
        grid=grid,
        in_specs=[
            pl.BlockSpec((blk, F), lambda i: (i, 0)),
            pl.BlockSpec((blk, 1), lambda i: (i, 0)),
            pl.BlockSpec((F, F), lambda i: (0, 0)),
        ],
        out_specs=pl.BlockSpec((blk, F), lambda i: (i, 0)),
        out_shape=jax.ShapeDtypeStruct((N_NODES, F), jnp.float32),
    )(h, norm, weight)


def _sc_body(x_hbm, edges_hbm, out_hbm,
             eidx, rows_v0, rows_v1, acc_sh, gsem0, gsem1, dsem0, dsem1):
    c = lax.axis_index("c")
    s = lax.axis_index("s")
    wid = c * NS + s
    start = wid * BASE_CHUNKS + jnp.minimum(wid, EXTRA)
    count = BASE_CHUNKS + jnp.where(wid < EXTRA, 1, 0)

    zv = jnp.zeros((16,), jnp.float32)

    def zrow(r, carry):
        for l in range(F // 16):
            rows_v0[r, pl.ds(l * 16, 16)] = zv
        return carry

    lax.fori_loop(0, K, zrow, 0)
    for z in range(ZCOPY):
        pltpu.sync_copy(rows_v0, acc_sh.at[pl.ds(s * ROWS_PER_TILE + z * K, K)])
    plsc.subcore_barrier()

    def idx_fetch(i, slot, dsem):
        pltpu.async_copy(
            edges_hbm.at[:, pl.ds((start + i) * K, K)], eidx.at[slot], dsem)

    def gather(slot, rows_v, gsem):
        pltpu.async_copy(x_hbm.at[eidx.at[slot, 0]], rows_v, gsem)

    idx_fetch(0, 0, dsem0)
    idx_fetch(1, 1, dsem1)
    pltpu.make_async_copy(
        edges_hbm.at[:, pl.ds(0, K)], eidx.at[0], dsem0).wait()
    gather(0, rows_v0, gsem0)
    pltpu.make_async_copy(
        edges_hbm.at[:, pl.ds(0, K)], eidx.at[1], dsem1).wait()
    gather(1, rows_v1, gsem1)

    def chunk(i, slot, rows_v, gsem, dsem):
        pltpu.make_async_copy(x_hbm.at[eidx.at[slot, 0]], rows_v, gsem).wait()
        pltpu.sync_copy(rows_v, acc_sh.at[eidx.at[slot, 1]], add=True)

        @pl.when(i + 2 < count)
        def _():
            idx_fetch(i + 2, slot, dsem)
            pltpu.make_async_copy(
                edges_hbm.at[:, pl.ds(0, K)], eidx.at[slot], dsem).wait()
            gather(slot, rows_v, gsem)

    def pair(j, carry):
        i = 2 * j
        chunk(i, 0, rows_v0, gsem0, dsem0)
        chunk(i + 1, 1, rows_v1, gsem1, dsem1)
        return carry

    lax.fori_loop(0, count // 2, pair, 0)

    @pl.when(count % 2 == 1)
    def _():
        chunk(count - 1, 0, rows_v0, gsem0, dsem0)
    plsc.subcore_barrier()

    pltpu.sync_copy(acc_sh.at[pl.ds(s * ROWS_PER_TILE, ROWS_PER_TILE)],
                    out_hbm.at[c, pl.ds(s * ROWS_PER_TILE, ROWS_PER_TILE)])


def _sc_aggregate(x, edges):
    mesh = plsc.VectorSubcoreMesh(
        core_axis_name="c", subcore_axis_name="s", num_cores=NC, num_subcores=NS)
    f = functools.partial(
        pl.kernel,
        out_type=jax.ShapeDtypeStruct((NC, N_PAD, F), jnp.float32),
        mesh=mesh,
        scratch_types=[
            pltpu.VMEM((2, 2, K), jnp.int32),
            pltpu.VMEM((K, F), jnp.float32),
            pltpu.VMEM((K, F), jnp.float32),
            pltpu.VMEM_SHARED((N_PAD, F), jnp.float32),
            pltpu.SemaphoreType.DMA,
            pltpu.SemaphoreType.DMA,
            pltpu.SemaphoreType.DMA,
            pltpu.SemaphoreType.DMA,
        ],
    )(_sc_body)
    return f(x, edges)


def _epilogue_body(p_ref, norm_ref, b_ref, o_ref):
    agg = p_ref[0] + p_ref[1]
    o_ref[...] = jnp.maximum(agg * norm_ref[...] + b_ref[...], 0.0)


def _tc_epilogue(partials, norm, bias2d):
    blk = 1000
    grid = (N_NODES // blk,)
    return pl.pallas_call(
        _epilogue_body,
        grid=grid,
        in_specs=[
            pl.BlockSpec((NC, blk, F), lambda i: (0, i, 0)),
            pl.BlockSpec((blk, 1), lambda i: (i, 0)),
            pl.BlockSpec((1, F), lambda i: (0, 0)),
        ],
        out_specs=pl.BlockSpec((blk, F), lambda i: (i, 0)),
        out_shape=jax.ShapeDtypeStruct((N_NODES, F), jnp.float32),
    )(partials, norm, bias2d)


def kernel(h, edge_index, norm, weight, bias):
    x = _tc_linear(h, norm, weight)
    partials = _sc_aggregate(x, edge_index)
    return _tc_epilogue(partials, norm, bias.reshape(1, F))


# speedup vs baseline: 1.1088x; 1.1088x over previous
import functools

import jax
import jax.numpy as jnp
from jax import lax
from jax.experimental import pallas as pl
from jax.experimental.pallas import tpu as pltpu
from jax.experimental.pallas import tpu_sc as plsc

N_NODES = 10000
N_EDGES = 320000
F = 128

NC = 2
NS = 16
NW = NC * NS
K = 128
TOTAL_CHUNKS = N_EDGES // K
BASE_CHUNKS = TOTAL_CHUNKS // NW
EXTRA = TOTAL_CHUNKS - BASE_CHUNKS * NW
N_PAD = 10240
ROWS_PER_TILE = N_PAD // NS
ZCOPY = ROWS_PER_TILE // K


def _linear_body(h_ref, norm_ref, w_ref, o_ref):
    o_ref[...] = norm_ref[...] * jnp.dot(
        h_ref[...], w_ref[...], preferred_element_type=jnp.float32)


def _tc_linear(h, norm, weight):
    blk = 2000
    grid = (N_NODES // blk,)
    return pl.pallas_call(
        _linear_body,
        grid=grid,
        in_specs=[
            pl.BlockSpec((blk, F), lambda i: (i, 0)),
            pl.BlockSpec((blk, 1), lambda i: (i, 0)),
            pl.BlockSpec((F, F), lambda i: (0, 0)),
        ],
        out_specs=pl.BlockSpec((blk, F), lambda i: (i, 0)),
        out_shape=jax.ShapeDtypeStruct((N_NODES, F), jnp.float32),
    )(h, norm, weight)


def _sc_body(x_hbm, edges_hbm, out_hbm,
             eidx, rows_v0, rows_v1, acc_sh, gsem0, gsem1, *dsems):
    c = lax.axis_index("c")
    s = lax.axis_index("s")
    wid = c * NS + s
    start = wid * BASE_CHUNKS + jnp.minimum(wid, EXTRA)
    count = BASE_CHUNKS + jnp.where(wid < EXTRA, 1, 0)

    zv = jnp.zeros((16,), jnp.float32)

    def zrow(r, carry):
        for l in range(F // 16):
            rows_v0[r, pl.ds(l * 16, 16)] = zv
        return carry

    lax.fori_loop(0, K, zrow, 0)
    for z in range(ZCOPY):
        pltpu.sync_copy(rows_v0, acc_sh.at[pl.ds(s * ROWS_PER_TILE + z * K, K)])
    plsc.subcore_barrier()

    rows = (rows_v0, rows_v1)
    gsems = (gsem0, gsem1)

    def idx_fetch(i, q):
        pltpu.async_copy(
            edges_hbm.at[:, pl.ds((start + i) * K, K)], eidx.at[q], dsems[q])

    def idx_wait(q):
        pltpu.make_async_copy(
            edges_hbm.at[:, pl.ds(0, K)], eidx.at[q], dsems[q]).wait()

    def gather(q, slot):
        pltpu.async_copy(x_hbm.at[eidx.at[q, 0]], rows[slot], gsems[slot])

    idx_fetch(0, 0)
    idx_fetch(1, 2)
    idx_fetch(2, 1)
    idx_fetch(3, 3)
    idx_wait(0)
    gather(0, 0)
    idx_wait(2)
    gather(2, 1)

    def chunk(i, slot, sub):
        q = slot * 2 + sub
        q2 = slot * 2 + (1 - sub)
        pltpu.make_async_copy(
            x_hbm.at[eidx.at[q, 0]], rows[slot], gsems[slot]).wait()
        pltpu.sync_copy(rows[slot], acc_sh.at[eidx.at[q, 1]], add=True)

        @pl.when(i + 4 < count)
        def _():
            idx_fetch(i + 4, q)

        @pl.when(i + 2 < count)
        def _():
            idx_wait(q2)
            gather(q2, slot)

    def pair(j, carry):
        i = 2 * j
        for p in (0, 1):
            @pl.when(j % 2 == p)
            def _():
                chunk(i, 0, p)
                chunk(i + 1, 1, p)
        return carry

    lax.fori_loop(0, count // 2, pair, 0)

    @pl.when(count % 2 == 1)
    def _():
        i = count - 1
        for p in (0, 1):
            @pl.when((i // 2) % 2 == p)
            def _():
                chunk(i, 0, p)
    plsc.subcore_barrier()

    pltpu.sync_copy(acc_sh.at[pl.ds(s * ROWS_PER_TILE, ROWS_PER_TILE)],
                    out_hbm.at[c, pl.ds(s * ROWS_PER_TILE, ROWS_PER_TILE)])


def _sc_aggregate(x, edges):
    mesh = plsc.VectorSubcoreMesh(
        core_axis_name="c", subcore_axis_name="s", num_cores=NC, num_subcores=NS)
    f = functools.partial(
        pl.kernel,
        out_type=jax.ShapeDtypeStruct((NC, N_PAD, F), jnp.float32),
        mesh=mesh,
        scratch_types=[
            pltpu.VMEM((4, 2, K), jnp.int32),
            pltpu.VMEM((K, F), jnp.float32),
            pltpu.VMEM((K, F), jnp.float32),
            pltpu.VMEM_SHARED((N_PAD, F), jnp.float32),
            pltpu.SemaphoreType.DMA,
            pltpu.SemaphoreType.DMA,
            pltpu.SemaphoreType.DMA,
            pltpu.SemaphoreType.DMA,
            pltpu.SemaphoreType.DMA,
            pltpu.SemaphoreType.DMA,
        ],
    )(_sc_body)
    return f(x, edges)


def _epilogue_body(p_ref, norm_ref, b_ref, o_ref):
    agg = p_ref[0] + p_ref[1]
    o_ref[...] = jnp.maximum(agg * norm_ref[...] + b_ref[...], 0.0)


def _tc_epilogue(partials, norm, bias2d):
    blk = 1000
    grid = (N_NODES // blk,)
    return pl.pallas_call(
        _epilogue_body,
        grid=grid,
        in_specs=[
            pl.BlockSpec((NC, blk, F), lambda i: (0, i, 0)),
            pl.BlockSpec((blk, 1), lambda i: (i, 0)),
            pl.BlockSpec((1, F), lambda i: (0, 0)),
        ],
        out_specs=pl.BlockSpec((blk, F), lambda i: (i, 0)),
        out_shape=jax.ShapeDtypeStruct((N_NODES, F), jnp.float32),
    )(partials, norm, bias2d)


def kernel(h, edge_index, norm, weight, bias):
    x = _tc_linear(h, norm, weight)
    partials = _sc_aggregate(x, edge_index)
    return _tc_epilogue(partials, norm, bias.reshape(1, F))
